# baseline (speedup 1.0000x reference)
import functools

import jax
import jax.numpy as jnp
from jax.experimental import pallas as pl
from jax.experimental.pallas import tpu as pltpu

_BK2 = 10000


def _nn_kernel(p_ref, c_ref, idx_ref, val_s, idx_s, *, bk2, num_rows, grid):
    i = pl.program_id(0)

    @pl.when(i == 0)
    def _init():
        val_s[...] = jnp.full_like(val_s, jnp.inf)
        idx_s[...] = jnp.zeros_like(idx_s)

    p = p_ref[...]
    np_ = p.shape[0]
    d = p.shape[1]
    c = c_ref[...]
    z = jnp.zeros((np_, d), jnp.float32)
    m2p = -2.0 * p
    wp = jnp.concatenate(
        [jnp.concatenate([m2p, z], axis=1),
         jnp.concatenate([z, m2p], axis=1)], axis=0)
    o = jnp.ones((np_, d), jnp.float32)
    zo = jnp.zeros((np_, d), jnp.float32)
    e = jnp.concatenate(
        [jnp.concatenate([o, zo], axis=1),
         jnp.concatenate([zo, o], axis=1)], axis=0)

    dots = jax.lax.dot_general(
        c, wp, (((1,), (1,)), ((), ())),
        preferred_element_type=jnp.float32)
    c2b = jax.lax.dot_general(
        c * c, e, (((1,), (1,)), ((), ())),
        preferred_element_type=jnp.float32)
    scores = dots + c2b

    local_min = jnp.min(scores, axis=0, keepdims=True)
    row_ids = jax.lax.broadcasted_iota(jnp.int32, scores.shape, 0)
    masked = jnp.where(scores == local_min, row_ids, num_rows)
    local_kk = jnp.min(masked, axis=0, keepdims=True)

    prev_v = val_s[...]
    prev_i = idx_s[...]
    better = local_min < prev_v
    val_s[...] = jnp.where(better, local_min, prev_v)
    idx_s[...] = jnp.where(better, i * bk2 + local_kk, prev_i)

    @pl.when(i == grid - 1)
    def _finish():
        v = val_s[...]
        ix = idx_s[...]
        ve, vo = v[:, :np_], v[:, np_:]
        ie = ix[:, :np_] * 2
        io = ix[:, np_:] * 2 + 1
        take_e = (ve < vo) | ((ve == vo) & (ie < io))
        idx_ref[...] = jnp.where(take_e, ie, io)


def kernel(prompt_embs, clip_embs):
    num_rows, d = clip_embs.shape
    p = prompt_embs.shape[0]
    bk2 = _BK2
    grid = (num_rows // 2) // bk2
    cpair = clip_embs.reshape(num_rows // 2, 2 * d)

    idx = pl.pallas_call(
        functools.partial(_nn_kernel, bk2=bk2, num_rows=num_rows, grid=grid),
        grid=(grid,),
        in_specs=[
            pl.BlockSpec((p, d), lambda i: (0, 0)),
            pl.BlockSpec((bk2, 2 * d), lambda i: (i, 0)),
        ],
        out_specs=pl.BlockSpec((1, p), lambda i: (0, 0)),
        out_shape=jax.ShapeDtypeStruct((1, p), jnp.int32),
        scratch_shapes=[
            pltpu.VMEM((1, 2 * p), jnp.float32),
            pltpu.VMEM((1, 2 * p), jnp.int32),
        ],
    )(prompt_embs, cpair)

    ids = idx[0, :]
    return (prompt_embs, prompt_embs, ids)
